# Initial kernel scaffold; baseline (speedup 1.0000x reference)
import jax
import jax.numpy as jnp
from jax.experimental import pallas as pl


def kernel(concat_embeddings, concat_type, pos_emb, tok_emb, ln_weight, ln_bias):
    raise NotImplementedError("write your pallas kernel here")


# speedup vs baseline: 4.9057x; 4.9057x over previous
import functools

import jax
import jax.numpy as jnp
from jax.experimental import pallas as pl
from jax.experimental.pallas import tpu as pltpu

B, S, D = 4, 8192, 768
EPS = 1e-12
BS = 512
NS = S // BS


def _body(x_ref, t_ref, pos_ref, tok_ref, w_ref, b_ref, o_ref):
    t = t_ref[0, 0, :]
    tok0 = tok_ref[0, :]
    tokd = tok_ref[1, :] - tok0
    x = x_ref[0] + pos_ref[...] + tok0[None, :] + t[:, None] * tokd[None, :]
    u = jnp.mean(x, axis=-1, keepdims=True)
    xc = x - u
    var = jnp.mean(xc * xc, axis=-1, keepdims=True)
    o_ref[0] = w_ref[...][None, :] * (xc * jax.lax.rsqrt(var + EPS)) + b_ref[...][None, :]


@jax.jit
def kernel(concat_embeddings, concat_type, pos_emb, tok_emb, ln_weight, ln_bias):
    t_f = concat_type.astype(jnp.float32).reshape(B * NS, 1, BS)
    grid = (NS, B)
    out = pl.pallas_call(
        _body,
        grid=grid,
        in_specs=[
            pl.BlockSpec((1, BS, D), lambda s, b: (b, s, 0)),
            pl.BlockSpec((1, 1, BS), lambda s, b: (b * NS + s, 0, 0)),
            pl.BlockSpec((BS, D), lambda s, b: (s, 0)),
            pl.BlockSpec((2, D), lambda s, b: (0, 0)),
            pl.BlockSpec((D,), lambda s, b: (0,)),
            pl.BlockSpec((D,), lambda s, b: (0,)),
        ],
        out_specs=pl.BlockSpec((1, BS, D), lambda s, b: (b, s, 0)),
        out_shape=jax.ShapeDtypeStruct((B, S, D), jnp.float32),
        compiler_params=pltpu.CompilerParams(
            dimension_semantics=("arbitrary", "arbitrary"),
        ),
    )(concat_embeddings, t_f, pos_emb, tok_emb, ln_weight, ln_bias)
    return out
